# baseline (speedup 1.0000x reference)
import functools

import jax
import jax.numpy as jnp
from jax import lax
from jax.experimental import pallas as pl
from jax.experimental.pallas import tpu as pltpu
from jax.experimental.pallas import tpu_sc as plsc

NC = 2
NS = 16
LANES = 16
NW = NC * NS
BATCH = 16384
D = 64
BPW = BATCH // NW
NUSER = 1000000
NITEM = 100000
CW = 128
NBUF = 8

USPLIT = 458752
TCBW = 8192
UT_FULL = NUSER // CW
UT_SC0 = USPLIT // CW
UT_SC = UT_FULL - UT_SC0
UT_BASE = UT_SC // NW
UT_EXTRA = UT_SC - UT_BASE * NW
UTAIL = NUSER - UT_FULL * CW
IT_FULL = NITEM // CW
IT_BASE = IT_FULL // NW
IT_EXTRA = IT_FULL - IT_BASE * NW
ITAIL = NITEM - IT_FULL * CW

_mesh = plsc.VectorSubcoreMesh(
    core_axis_name="c", subcore_axis_name="s", num_cores=NC, num_subcores=NS
)


def _dot_chunk(buf, row0, w_v, w_off, width, out_ref, out_off):
    ngrp = width // LANES
    gblk = min(4, ngrp)
    wvecs = [w_v[pl.ds(w_off + k * LANES, LANES)] for k in range(D // LANES)]
    for gb0 in range(0, ngrp, gblk):
        nb = min(gblk, ngrp - gb0)
        accs = [None] * nb
        for cb in range(D // LANES):
            spl = [lax.broadcast(wvecs[cb][j], (LANES,)) for j in range(LANES)]
            for g in range(nb):
                grp = gb0 + g
                a = accs[g]
                for j in range(LANES):
                    c = cb * LANES + j
                    p = buf[row0 + c, pl.ds(grp * LANES, LANES)] * spl[j]
                    a = p if a is None else a + p
                accs[g] = a
        for g in range(nb):
            out_ref[pl.ds(out_off + (gb0 + g) * LANES, LANES)] = accs[g]


@functools.partial(
    pl.kernel,
    out_type=(
        jax.ShapeDtypeStruct((NUSER,), jnp.float32),
        jax.ShapeDtypeStruct((NITEM,), jnp.float32),
    ),
    mesh=_mesh,
    scratch_types=[
        pltpu.VMEM((136,), jnp.float32),
        pltpu.VMEM((NBUF * D, CW), jnp.float32),
        pltpu.VMEM(((UT_BASE + 1) * CW,), jnp.float32),
        pltpu.VMEM(((IT_BASE + 1) * CW,), jnp.float32),
        pltpu.VMEM((D, UTAIL), jnp.float32),
        pltpu.VMEM((D, ITAIL), jnp.float32),
        pltpu.VMEM((UTAIL,), jnp.float32),
        pltpu.VMEM((ITAIL,), jnp.float32),
        pltpu.SemaphoreType.DMA,
    ],
)
def _sweep(tu_hbm, ti_hbm, w_hbm, su_hbm, si_hbm,
           w_v, ring, s_uv, s_iv, tb_u, tb_i, ts_u, ts_i, sem):
    wid = lax.axis_index("s") * NC + lax.axis_index("c")

    pltpu.sync_copy(w_hbm, w_v)

    def sweep_table(t_hbm, w_off, start, total, s_v):
        def fire(t, slot):
            off = pl.multiple_of(t * CW, CW)
            row = pl.multiple_of(slot * D, D)
            pltpu.async_copy(
                t_hbm.at[:, pl.ds(off, CW)], ring.at[pl.ds(row, D)], sem)

        def drain():
            pltpu.make_async_copy(
                t_hbm.at[:, pl.ds(0, CW)], ring.at[pl.ds(0, D)], sem).wait()

        pre = NBUF - 1
        for k in range(pre):
            @pl.when(k < total)
            def _(k=k):
                fire(start + k, k)

        def body(t, carry):
            slot = lax.rem(t, NBUF)
            row0 = pl.multiple_of(slot * D, D)
            drain()

            @pl.when(t + pre < total)
            def _():
                fire(start + t + pre, lax.rem(t + pre, NBUF))

            _dot_chunk(ring, row0, w_v, w_off, CW, s_v, t * CW)
            return carry

        lax.fori_loop(0, total, body, 0)

    ustart = UT_SC0 + wid * UT_BASE + jnp.minimum(wid, UT_EXTRA)
    utotal = UT_BASE + (wid < UT_EXTRA).astype(jnp.int32)
    sweep_table(tu_hbm, 0, ustart, utotal, s_uv)
    pltpu.sync_copy(s_uv.at[pl.ds(0, UT_BASE * CW)],
                    su_hbm.at[pl.ds(ustart * CW, UT_BASE * CW)])

    @pl.when(wid < UT_EXTRA)
    def _():
        pltpu.sync_copy(
            s_uv.at[pl.ds(UT_BASE * CW, CW)],
            su_hbm.at[pl.ds(ustart * CW + UT_BASE * CW, CW)])

    istart = wid * IT_BASE + jnp.minimum(wid, IT_EXTRA)
    itotal = IT_BASE + (wid < IT_EXTRA).astype(jnp.int32)
    sweep_table(ti_hbm, D, istart, itotal, s_iv)
    pltpu.sync_copy(s_iv.at[pl.ds(0, IT_BASE * CW)],
                    si_hbm.at[pl.ds(istart * CW, IT_BASE * CW)])

    @pl.when(wid < IT_EXTRA)
    def _():
        pltpu.sync_copy(
            s_iv.at[pl.ds(IT_BASE * CW, CW)],
            si_hbm.at[pl.ds(istart * CW + IT_BASE * CW, CW)])

    @pl.when(wid == NW - 1)
    def _():
        pltpu.sync_copy(tu_hbm.at[:, pl.ds(UT_FULL * CW, UTAIL)], tb_u)
        _dot_chunk(tb_u, 0, w_v, 0, UTAIL, ts_u, 0)
        pltpu.sync_copy(ts_u, su_hbm.at[pl.ds(UT_FULL * CW, UTAIL)])
        pltpu.sync_copy(ti_hbm.at[:, pl.ds(IT_FULL * CW, ITAIL)], tb_i)
        _dot_chunk(tb_i, 0, w_v, D, ITAIL, ts_i, 0)
        pltpu.sync_copy(ts_i, si_hbm.at[pl.ds(IT_FULL * CW, ITAIL)])


@functools.partial(
    pl.kernel,
    out_type=jax.ShapeDtypeStruct((BATCH,), jnp.float32),
    mesh=_mesh,
    scratch_types=[
        pltpu.VMEM((4, 128), jnp.int32),
        pltpu.VMEM((4, 128), jnp.int32),
        pltpu.VMEM((4, 128), jnp.int32),
        pltpu.VMEM((BPW,), jnp.float32),
        pltpu.VMEM((BPW,), jnp.float32),
        pltpu.VMEM((BPW,), jnp.float32),
        pltpu.VMEM((LANES,), jnp.float32),
        pltpu.VMEM((BPW,), jnp.float32),
        pltpu.SemaphoreType.DMA,
        pltpu.SemaphoreType.DMA,
    ],
    compiler_params=pltpu.CompilerParams(
        needs_layout_passes=False, use_tc_tiling_on_sc=False),
)
def _gather_out(su_hbm, slo_hbm, si_hbm, uid_hbm, ulo_hbm, iid_hbm, bv_hbm,
                out_hbm, idx_u, idx_lo, idx_i, g_hi, g_lo, g_i, b_v, out_v,
                sem_u, sem_i):
    wid = lax.axis_index("s") * NC + lax.axis_index("c")
    base = wid * BPW

    pltpu.sync_copy(uid_hbm.at[pl.ds(wid * 4, 4)], idx_u)
    pltpu.sync_copy(ulo_hbm.at[pl.ds(wid * 4, 4)], idx_lo)
    pltpu.sync_copy(iid_hbm.at[pl.ds(wid * 4, 4)], idx_i)
    pltpu.sync_copy(bv_hbm, b_v)

    copies = []
    for j in range(4):
        copies.append(pltpu.async_copy(
            su_hbm.at[idx_u.at[j]], g_hi.at[pl.ds(j * 128, 128)], sem_u))
        copies.append(pltpu.async_copy(
            slo_hbm.at[idx_lo.at[j]], g_lo.at[pl.ds(j * 128, 128)], sem_u))
        copies.append(pltpu.async_copy(
            si_hbm.at[idx_i.at[j]], g_i.at[pl.ds(j * 128, 128)], sem_i))
    for c in copies:
        c.wait()

    bias = b_v[...]

    def body(g, carry):
        b0 = g * LANES
        j = b0 // 128
        k = b0 % 128
        idx = idx_u[j, pl.ds(k, LANES)]
        s_user = jnp.where(idx < USPLIT,
                           g_lo[pl.ds(b0, LANES)], g_hi[pl.ds(b0, LANES)])
        out_v[pl.ds(b0, LANES)] = s_user + g_i[pl.ds(b0, LANES)] + bias
        return carry

    for g in range(BPW // LANES):
        body(g, 0)

    pltpu.sync_copy(out_v, out_hbm.at[pl.ds(base, BPW)])


def _tc_matvec_body(x_ref, w_ref, o_ref):
    o_ref[...] = jnp.dot(w_ref[...], x_ref[...],
                         preferred_element_type=jnp.float32)


_tc_matvec = pl.pallas_call(
    _tc_matvec_body,
    grid=(USPLIT // TCBW,),
    in_specs=[
        pl.BlockSpec((D, TCBW), lambda i: (0, i)),
        pl.BlockSpec((1, D), lambda i: (0, 0)),
    ],
    out_specs=pl.BlockSpec((1, TCBW), lambda i: (0, i)),
    out_shape=jax.ShapeDtypeStruct((1, USPLIT), jnp.float32),
)


def kernel(user_ids, item_ids, user_table, item_table, fc_w, fc_b):
    t_u = user_table.T
    t_i = item_table.T
    w_pad = jnp.concatenate(
        [fc_w.reshape(-1), jnp.zeros((8,), jnp.float32)])
    s_u, s_i = _sweep(t_u, t_i, w_pad)
    wu_row = fc_w[:, :D]
    s_lo = _tc_matvec(t_u, wu_row).reshape(-1)
    uid = user_ids.astype(jnp.int32)
    ulo = jnp.remainder(uid, USPLIT).reshape(NW * 4, 128)
    uid = uid.reshape(NW * 4, 128)
    iid = item_ids.astype(jnp.int32).reshape(NW * 4, 128)
    bv = jnp.full((LANES,), fc_b[0], jnp.float32)
    return _gather_out(s_u, s_lo, s_i, uid, ulo, iid, bv)


# speedup vs baseline: 1.0407x; 1.0407x over previous
import functools

import jax
import jax.numpy as jnp
from jax import lax
from jax.experimental import pallas as pl
from jax.experimental.pallas import tpu as pltpu
from jax.experimental.pallas import tpu_sc as plsc

NC = 2
NS = 16
LANES = 16
NW = NC * NS
BATCH = 16384
D = 64
BPW = BATCH // NW
NUSER = 1000000
NITEM = 100000
CW = 128
NBUF = 8

USPLIT = 524288
TCBW = 8192
UT_FULL = NUSER // CW
UT_SC0 = USPLIT // CW
UT_SC = UT_FULL - UT_SC0
UT_BASE = UT_SC // NW
UT_EXTRA = UT_SC - UT_BASE * NW
UTAIL = NUSER - UT_FULL * CW
IT_FULL = NITEM // CW
IT_BASE = IT_FULL // NW
IT_EXTRA = IT_FULL - IT_BASE * NW
ITAIL = NITEM - IT_FULL * CW

_mesh = plsc.VectorSubcoreMesh(
    core_axis_name="c", subcore_axis_name="s", num_cores=NC, num_subcores=NS
)


def _dot_chunk(buf, row0, w_v, w_off, width, out_ref, out_off):
    ngrp = width // LANES
    gblk = min(4, ngrp)
    wvecs = [w_v[pl.ds(w_off + k * LANES, LANES)] for k in range(D // LANES)]
    for gb0 in range(0, ngrp, gblk):
        nb = min(gblk, ngrp - gb0)
        accs = [None] * nb
        for cb in range(D // LANES):
            spl = [lax.broadcast(wvecs[cb][j], (LANES,)) for j in range(LANES)]
            for g in range(nb):
                grp = gb0 + g
                a = accs[g]
                for j in range(LANES):
                    c = cb * LANES + j
                    p = buf[row0 + c, pl.ds(grp * LANES, LANES)] * spl[j]
                    a = p if a is None else a + p
                accs[g] = a
        for g in range(nb):
            out_ref[pl.ds(out_off + (gb0 + g) * LANES, LANES)] = accs[g]


@functools.partial(
    pl.kernel,
    out_type=(
        jax.ShapeDtypeStruct((NUSER,), jnp.float32),
        jax.ShapeDtypeStruct((NITEM,), jnp.float32),
    ),
    mesh=_mesh,
    scratch_types=[
        pltpu.VMEM((136,), jnp.float32),
        pltpu.VMEM((NBUF * D, CW), jnp.float32),
        pltpu.VMEM(((UT_BASE + 1) * CW,), jnp.float32),
        pltpu.VMEM(((IT_BASE + 1) * CW,), jnp.float32),
        pltpu.VMEM((D, UTAIL), jnp.float32),
        pltpu.VMEM((D, ITAIL), jnp.float32),
        pltpu.VMEM((UTAIL,), jnp.float32),
        pltpu.VMEM((ITAIL,), jnp.float32),
        pltpu.SemaphoreType.DMA,
    ],
)
def _sweep(tu_hbm, ti_hbm, w_hbm, su_hbm, si_hbm,
           w_v, ring, s_uv, s_iv, tb_u, tb_i, ts_u, ts_i, sem):
    wid = lax.axis_index("s") * NC + lax.axis_index("c")

    pltpu.sync_copy(w_hbm, w_v)

    def sweep_table(t_hbm, w_off, start, total, s_v):
        def fire(t, slot):
            off = pl.multiple_of(t * CW, CW)
            row = pl.multiple_of(slot * D, D)
            pltpu.async_copy(
                t_hbm.at[:, pl.ds(off, CW)], ring.at[pl.ds(row, D)], sem)

        def drain():
            pltpu.make_async_copy(
                t_hbm.at[:, pl.ds(0, CW)], ring.at[pl.ds(0, D)], sem).wait()

        pre = NBUF - 1
        for k in range(pre):
            @pl.when(k < total)
            def _(k=k):
                fire(start + k, k)

        def body(t, carry):
            slot = lax.rem(t, NBUF)
            row0 = pl.multiple_of(slot * D, D)
            drain()

            @pl.when(t + pre < total)
            def _():
                fire(start + t + pre, lax.rem(t + pre, NBUF))

            _dot_chunk(ring, row0, w_v, w_off, CW, s_v, t * CW)
            return carry

        lax.fori_loop(0, total, body, 0)

    ustart = UT_SC0 + wid * UT_BASE + jnp.minimum(wid, UT_EXTRA)
    utotal = UT_BASE + (wid < UT_EXTRA).astype(jnp.int32)
    sweep_table(tu_hbm, 0, ustart, utotal, s_uv)
    pltpu.sync_copy(s_uv.at[pl.ds(0, UT_BASE * CW)],
                    su_hbm.at[pl.ds(ustart * CW, UT_BASE * CW)])

    @pl.when(wid < UT_EXTRA)
    def _():
        pltpu.sync_copy(
            s_uv.at[pl.ds(UT_BASE * CW, CW)],
            su_hbm.at[pl.ds(ustart * CW + UT_BASE * CW, CW)])

    istart = wid * IT_BASE + jnp.minimum(wid, IT_EXTRA)
    itotal = IT_BASE + (wid < IT_EXTRA).astype(jnp.int32)
    sweep_table(ti_hbm, D, istart, itotal, s_iv)
    pltpu.sync_copy(s_iv.at[pl.ds(0, IT_BASE * CW)],
                    si_hbm.at[pl.ds(istart * CW, IT_BASE * CW)])

    @pl.when(wid < IT_EXTRA)
    def _():
        pltpu.sync_copy(
            s_iv.at[pl.ds(IT_BASE * CW, CW)],
            si_hbm.at[pl.ds(istart * CW + IT_BASE * CW, CW)])

    @pl.when(wid == NW - 1)
    def _():
        pltpu.sync_copy(tu_hbm.at[:, pl.ds(UT_FULL * CW, UTAIL)], tb_u)
        _dot_chunk(tb_u, 0, w_v, 0, UTAIL, ts_u, 0)
        pltpu.sync_copy(ts_u, su_hbm.at[pl.ds(UT_FULL * CW, UTAIL)])
        pltpu.sync_copy(ti_hbm.at[:, pl.ds(IT_FULL * CW, ITAIL)], tb_i)
        _dot_chunk(tb_i, 0, w_v, D, ITAIL, ts_i, 0)
        pltpu.sync_copy(ts_i, si_hbm.at[pl.ds(IT_FULL * CW, ITAIL)])


@functools.partial(
    pl.kernel,
    out_type=jax.ShapeDtypeStruct((BATCH,), jnp.float32),
    mesh=_mesh,
    scratch_types=[
        pltpu.VMEM((4, 128), jnp.int32),
        pltpu.VMEM((4, 128), jnp.int32),
        pltpu.VMEM((4, 128), jnp.int32),
        pltpu.VMEM((BPW,), jnp.float32),
        pltpu.VMEM((BPW,), jnp.float32),
        pltpu.VMEM((BPW,), jnp.float32),
        pltpu.VMEM((LANES,), jnp.float32),
        pltpu.VMEM((BPW,), jnp.float32),
        pltpu.SemaphoreType.DMA,
        pltpu.SemaphoreType.DMA,
    ],
    compiler_params=pltpu.CompilerParams(
        needs_layout_passes=False, use_tc_tiling_on_sc=False),
)
def _gather_out(su_hbm, slo_hbm, si_hbm, uid_hbm, ulo_hbm, iid_hbm, bv_hbm,
                out_hbm, idx_u, idx_lo, idx_i, g_hi, g_lo, g_i, b_v, out_v,
                sem_u, sem_i):
    wid = lax.axis_index("s") * NC + lax.axis_index("c")
    base = wid * BPW

    pltpu.sync_copy(uid_hbm.at[pl.ds(wid * 4, 4)], idx_u)
    pltpu.sync_copy(ulo_hbm.at[pl.ds(wid * 4, 4)], idx_lo)
    pltpu.sync_copy(iid_hbm.at[pl.ds(wid * 4, 4)], idx_i)
    pltpu.sync_copy(bv_hbm, b_v)

    copies = []
    for j in range(4):
        copies.append(pltpu.async_copy(
            su_hbm.at[idx_u.at[j]], g_hi.at[pl.ds(j * 128, 128)], sem_u))
        copies.append(pltpu.async_copy(
            slo_hbm.at[idx_lo.at[j]], g_lo.at[pl.ds(j * 128, 128)], sem_u))
        copies.append(pltpu.async_copy(
            si_hbm.at[idx_i.at[j]], g_i.at[pl.ds(j * 128, 128)], sem_i))
    for c in copies:
        c.wait()

    bias = b_v[...]

    def body(g, carry):
        b0 = g * LANES
        j = b0 // 128
        k = b0 % 128
        idx = idx_u[j, pl.ds(k, LANES)]
        s_user = jnp.where(idx < USPLIT,
                           g_lo[pl.ds(b0, LANES)], g_hi[pl.ds(b0, LANES)])
        out_v[pl.ds(b0, LANES)] = s_user + g_i[pl.ds(b0, LANES)] + bias
        return carry

    for g in range(BPW // LANES):
        body(g, 0)

    pltpu.sync_copy(out_v, out_hbm.at[pl.ds(base, BPW)])


def _tc_matvec_body(x_ref, w_ref, o_ref):
    o_ref[...] = jnp.dot(w_ref[...], x_ref[...],
                         preferred_element_type=jnp.float32)[0]


_tc_matvec = pl.pallas_call(
    _tc_matvec_body,
    grid=(USPLIT // TCBW,),
    in_specs=[
        pl.BlockSpec((D, TCBW), lambda i: (0, i)),
        pl.BlockSpec((1, D), lambda i: (0, 0)),
    ],
    out_specs=pl.BlockSpec((TCBW,), lambda i: (i,)),
    out_shape=jax.ShapeDtypeStruct((USPLIT,), jnp.float32),
)


def kernel(user_ids, item_ids, user_table, item_table, fc_w, fc_b):
    t_u = user_table.T
    t_i = item_table.T
    w_pad = jnp.concatenate(
        [fc_w.reshape(-1), jnp.zeros((8,), jnp.float32)])
    s_u, s_i = _sweep(t_u, t_i, w_pad)
    wu_row = fc_w[:, :D]
    s_lo = _tc_matvec(t_u, wu_row)
    uid = user_ids.astype(jnp.int32)
    ulo = jnp.remainder(uid, USPLIT).reshape(NW * 4, 128)
    uid = uid.reshape(NW * 4, 128)
    iid = item_ids.astype(jnp.int32).reshape(NW * 4, 128)
    bv = jnp.full((LANES,), fc_b[0], jnp.float32)
    return _gather_out(s_u, s_lo, s_i, uid, ulo, iid, bv)
